# baseline (speedup 1.0000x reference)
import functools

import jax
import jax.numpy as jnp
from jax import lax
from jax.experimental import pallas as pl
from jax.experimental.pallas import tpu as pltpu
from jax.experimental.pallas import tpu_sc as plsc

N = 16384
D = 64
MAX_DEPTH = 50
DROWS = MAX_DEPTH + 1
L = 16
NC, NS = 2, 16
NW = NC * NS
BPW = N // NW
CH = 128
NCH = BPW // CH
GPC = CH // L
NGRP = BPW // L

_mesh = plsc.VectorSubcoreMesh(core_axis_name="c", subcore_axis_name="s")
_params = pltpu.CompilerParams(use_tc_tiling_on_sc=False)


@functools.partial(
    pl.kernel,
    mesh=_mesh,
    compiler_params=_params,
    out_type=jax.ShapeDtypeStruct((N, D), jnp.float32),
    scratch_types=[
        pltpu.VMEM((NCH, CH), jnp.int32),
        pltpu.VMEM((NCH, CH), jnp.int32),
        pltpu.VMEM((BPW, D), jnp.float32),
        pltpu.VMEM((BPW, D), jnp.float32),
        pltpu.SemaphoreType.DMA,
        pltpu.SemaphoreType.DMA,
    ],
)
def _encode_type(xn_hbm, depth_hbm, ttab, dtab_rep, out_hbm,
                 idx_t, idx_d, rows_t, rows_d, sem0, sem1):
    wid = lax.axis_index("s") * NC + lax.axis_index("c")
    base = wid * BPW

    idx_copies = []
    for j in range(NCH):
        toff = (wid * NCH + j) * (2 * CH)
        idx_copies.append(pltpu.async_copy(xn_hbm.at[pl.ds(toff, CH)],
                                           idx_t.at[j], sem0))
        idx_copies.append(pltpu.async_copy(depth_hbm.at[pl.ds(base + j * CH, CH)],
                                           idx_d.at[j], sem0))
    for c in idx_copies:
        c.wait()

    for j in range(NCH):
        for i in range(GPC):
            s = pl.ds(i * L, L)
            idx_d[j, s] = jnp.minimum(idx_d[j, s], MAX_DEPTH) + wid * DROWS

    copies = []
    for j in range(NCH):
        copies.append(pltpu.async_copy(ttab.at[idx_t.at[j]],
                                       rows_t.at[pl.ds(j * CH, CH)], sem1))
        copies.append(pltpu.async_copy(dtab_rep.at[idx_d.at[j]],
                                       rows_d.at[pl.ds(j * CH, CH)], sem1))
    for c in copies:
        c.wait()

    @pl.loop(0, BPW)
    def _acc(g):
        for c in range(D // L):
            s = pl.ds(c * L, L)
            rows_t[g, s] = rows_t[g, s] + rows_d[g, s]

    pltpu.sync_copy(rows_t, out_hbm.at[pl.ds(base, BPW)])


@functools.partial(
    pl.kernel,
    mesh=_mesh,
    compiler_params=_params,
    out_type=jax.ShapeDtypeStruct((N, D), jnp.float32),
    scratch_types=[
        pltpu.VMEM((NCH, CH), jnp.int32),
        pltpu.VMEM((BPW, D), jnp.float32),
        pltpu.VMEM((BPW, D), jnp.float32),
        pltpu.SemaphoreType.DMA,
        pltpu.SemaphoreType.DMA,
    ],
)
def _encode_attr(xn_hbm, part_hbm, atab, out_hbm,
                 idx_a, acc, rows_a, sem0, sem1):
    wid = lax.axis_index("s") * NC + lax.axis_index("c")
    base = wid * BPW

    copies = []
    for j in range(NCH):
        aoff = (wid * NCH + j) * (2 * CH) + CH
        pltpu.sync_copy(xn_hbm.at[pl.ds(aoff, CH)], idx_a.at[j])
        copies.append(pltpu.async_copy(atab.at[idx_a.at[j]],
                                       rows_a.at[pl.ds(j * CH, CH)], sem1))
    pc = pltpu.async_copy(part_hbm.at[pl.ds(base, BPW)], acc, sem0)
    pc.wait()
    for c in copies:
        c.wait()

    @pl.loop(0, BPW)
    def _acc_loop(g):
        for c in range(D // L):
            s = pl.ds(c * L, L)
            acc[g, s] = acc[g, s] + rows_a[g, s]

    pltpu.sync_copy(acc, out_hbm.at[pl.ds(base, BPW)])


def kernel(x, depth, type_table, attr_table, depth_table):
    xn = x.astype(jnp.int32).reshape(N // CH, CH, 2).transpose(0, 2, 1).reshape(-1)
    dtab_rep = jnp.tile(depth_table, (NW, 1))
    part = _encode_type(xn, depth.astype(jnp.int32), type_table, dtab_rep)
    return _encode_attr(xn, part, attr_table)


# speedup vs baseline: 1.0568x; 1.0568x over previous
import functools

import jax
import jax.numpy as jnp
from jax import lax
from jax.experimental import pallas as pl
from jax.experimental.pallas import tpu as pltpu
from jax.experimental.pallas import tpu_sc as plsc

N = 16384
D = 64
MAX_DEPTH = 50
DROWS = MAX_DEPTH + 1
L = 16
NC, NS = 2, 16
NW = NC * NS
BPW = N // NW
CH = 128
NCH = BPW // CH
GPC = CH // L
NGRP = BPW // L

_mesh = plsc.VectorSubcoreMesh(core_axis_name="c", subcore_axis_name="s")
_params = pltpu.CompilerParams(use_tc_tiling_on_sc=False)


@functools.partial(
    pl.kernel,
    mesh=_mesh,
    compiler_params=_params,
    out_type=jax.ShapeDtypeStruct((N, D), jnp.float32),
    scratch_types=[
        pltpu.VMEM((NCH, CH), jnp.int32),
        pltpu.VMEM((NCH, CH), jnp.int32),
        pltpu.VMEM((DROWS, D), jnp.float32),
        pltpu.VMEM((BPW, D), jnp.float32),
        pltpu.SemaphoreType.DMA,
        pltpu.SemaphoreType.DMA,
    ],
)
def _encode_type(xn_hbm, depth_hbm, ttab, dtab, out_hbm,
                 idx_t, idx_d, dtab_v, rows_t, sem0, sem1):
    wid = lax.axis_index("s") * NC + lax.axis_index("c")
    base = wid * BPW

    idx_copies = []
    for j in range(NCH):
        toff = (wid * NCH + j) * (2 * CH)
        idx_copies.append(pltpu.async_copy(xn_hbm.at[pl.ds(toff, CH)],
                                           idx_t.at[j], sem0))
        idx_copies.append(pltpu.async_copy(depth_hbm.at[pl.ds(base + j * CH, CH)],
                                           idx_d.at[j], sem0))
    pltpu.sync_copy(dtab, dtab_v)
    for c in idx_copies:
        c.wait()

    for j in range(NCH):
        for i in range(GPC):
            s = pl.ds(i * L, L)
            idx_d[j, s] = jnp.minimum(idx_d[j, s], MAX_DEPTH)

    copies = []
    for j in range(NCH):
        copies.append(pltpu.async_copy(ttab.at[idx_t.at[j]],
                                       rows_t.at[pl.ds(j * CH, CH)], sem1))
    for c in copies:
        c.wait()

    @pl.loop(0, NGRP)
    def _acc(g):
        dvec = idx_d[g // GPC, pl.ds((g % GPC) * L, L)]
        for l in range(L):
            d = dvec[l]
            row = g * L + l
            for c in range(D // L):
                s = pl.ds(c * L, L)
                rows_t[row, s] = rows_t[row, s] + dtab_v[d, s]

    pltpu.sync_copy(rows_t, out_hbm.at[pl.ds(base, BPW)])


@functools.partial(
    pl.kernel,
    mesh=_mesh,
    compiler_params=_params,
    out_type=jax.ShapeDtypeStruct((N, D), jnp.float32),
    scratch_types=[
        pltpu.VMEM((NCH, CH), jnp.int32),
        pltpu.VMEM((BPW, D), jnp.float32),
        pltpu.VMEM((BPW, D), jnp.float32),
        pltpu.SemaphoreType.DMA,
        pltpu.SemaphoreType.DMA,
    ],
)
def _encode_attr(xn_hbm, part_hbm, atab, out_hbm,
                 idx_a, acc, rows_a, sem0, sem1):
    wid = lax.axis_index("s") * NC + lax.axis_index("c")
    base = wid * BPW

    icopies = []
    for j in range(NCH):
        aoff = (wid * NCH + j) * (2 * CH) + CH
        icopies.append(pltpu.async_copy(xn_hbm.at[pl.ds(aoff, CH)],
                                        idx_a.at[j], sem0))
    pc = pltpu.async_copy(part_hbm.at[pl.ds(base, BPW)], acc, sem0)
    for c in icopies:
        c.wait()
    copies = []
    for j in range(NCH):
        copies.append(pltpu.async_copy(atab.at[idx_a.at[j]],
                                       rows_a.at[pl.ds(j * CH, CH)], sem1))
    pc.wait()
    for c in copies:
        c.wait()

    @pl.loop(0, BPW)
    def _acc_loop(g):
        for c in range(D // L):
            s = pl.ds(c * L, L)
            acc[g, s] = acc[g, s] + rows_a[g, s]

    pltpu.sync_copy(acc, out_hbm.at[pl.ds(base, BPW)])


def kernel(x, depth, type_table, attr_table, depth_table):
    xn = x.astype(jnp.int32).reshape(N // CH, CH, 2).transpose(0, 2, 1).reshape(-1)
    part = _encode_type(xn, depth.astype(jnp.int32), type_table, depth_table)
    return _encode_attr(xn, part, attr_table)
